# Initial kernel scaffold; baseline (speedup 1.0000x reference)
import jax
import jax.numpy as jnp
from jax.experimental import pallas as pl


def kernel(x, edge_index, W, b, gamma, beta):
    raise NotImplementedError("write your pallas kernel here")


# speedup vs baseline: 25.1652x; 25.1652x over previous
import functools

import jax
import jax.numpy as jnp
from jax import lax
from jax.experimental import pallas as pl
from jax.experimental.pallas import tpu as pltpu
from jax.experimental.pallas import tpu_sc as plsc

N = 10000
D = 128
E = 320000
EPS = 1e-5

NC, NS = 2, 16
NW = NC * NS
N_EXT = 10240
N_PAD_ROWS = 64
E_PAD = 327680
CHUNK = 128
K = 8
EPT_ROWS = E_PAD // NW // CHUNK
NBLK = EPT_ROWS // K
RPT = N_EXT // NS

_mesh = lambda: plsc.VectorSubcoreMesh(core_axis_name="c", subcore_axis_name="s")


@functools.partial(
    pl.kernel,
    out_type=jax.ShapeDtypeStruct((NC, N_EXT), jnp.float32),
    mesh=_mesh(),
    scratch_types=[
        pltpu.VMEM((K, CHUNK), jnp.int32),
        pltpu.VMEM((CHUNK,), jnp.float32),
        pltpu.VMEM((RPT,), jnp.float32),
        pltpu.VMEM_SHARED((N_EXT,), jnp.float32),
    ],
)
def _deg_kernel(dst_hbm, degp_hbm, dst_v, ones_v, z_v, deg_sh):
    cid = lax.axis_index("c")
    sid = lax.axis_index("s")
    wid = sid * NC + cid
    for i in range(CHUNK // 16):
        ones_v[pl.ds(i * 16, 16)] = jnp.ones((16,), jnp.float32)
    for i in range(RPT // 16):
        z_v[pl.ds(i * 16, 16)] = jnp.zeros((16,), jnp.float32)
    pltpu.sync_copy(z_v, deg_sh.at[pl.ds(sid * RPT, RPT)])
    plsc.subcore_barrier()

    def blk(b, carry):
        row0 = wid * EPT_ROWS + b * K
        pltpu.sync_copy(dst_hbm.at[pl.ds(row0, K)], dst_v)
        for j in range(K):
            pltpu.sync_copy(ones_v, deg_sh.at[dst_v.at[j]], add=True)
        return carry

    lax.fori_loop(0, NBLK, blk, 0)
    plsc.subcore_barrier()
    pltpu.sync_copy(deg_sh.at[pl.ds(sid * RPT, RPT)],
                    degp_hbm.at[cid, pl.ds(sid * RPT, RPT)])


@functools.partial(
    pl.kernel,
    out_type=jax.ShapeDtypeStruct((NC, N_EXT, D), jnp.float32),
    mesh=_mesh(),
    scratch_types=[
        pltpu.VMEM((K, CHUNK), jnp.int32),
        pltpu.VMEM((K, CHUNK), jnp.int32),
        pltpu.VMEM((CHUNK, D), jnp.float32),
        pltpu.VMEM_SHARED((N_EXT, D), jnp.float32),
    ],
)
def _scatter_kernel(src_hbm, dst_hbm, y_hbm, acc_hbm, src_v, dst_v, rows_v, acc_sh):
    cid = lax.axis_index("c")
    sid = lax.axis_index("s")
    wid = sid * NC + cid
    pltpu.sync_copy(y_hbm.at[pl.ds(sid * RPT, RPT)],
                    acc_sh.at[pl.ds(sid * RPT, RPT)])
    plsc.subcore_barrier()

    def blk(b, carry):
        row0 = wid * EPT_ROWS + b * K
        pltpu.sync_copy(src_hbm.at[pl.ds(row0, K)], src_v)
        pltpu.sync_copy(dst_hbm.at[pl.ds(row0, K)], dst_v)
        for j in range(K):
            pltpu.sync_copy(y_hbm.at[src_v.at[j]], rows_v)
            pltpu.sync_copy(rows_v, acc_sh.at[dst_v.at[j]], add=True)
        return carry

    lax.fori_loop(0, NBLK, blk, 0)
    plsc.subcore_barrier()
    pltpu.sync_copy(acc_sh.at[pl.ds(sid * RPT, RPT)],
                    acc_hbm.at[cid, pl.ds(sid * RPT, RPT)])


RB = 1024
RBN = 1000


def _prep_body(x_ref, w_ref, d0_ref, d1_ref, y_ref, dinv_ref):
    deg = d0_ref[...] + d1_ref[...] + 1.0
    dinv = lax.rsqrt(deg)
    xw = jnp.dot(x_ref[...], w_ref[...], preferred_element_type=jnp.float32)
    y_ref[...] = xw * dinv
    dinv_ref[...] = dinv


def _stats_body(p0_ref, p1_ref, y_ref, dinv_ref, prm_ref, agg_ref, st_ref):
    i = pl.program_id(0)
    s = p0_ref[0] + p1_ref[0] - y_ref[...]
    b = prm_ref[0:1, :]
    agg = s * dinv_ref[...] + b
    agg_ref[...] = agg
    su = jnp.sum(agg, axis=0, keepdims=True)
    sq = jnp.sum(agg * agg, axis=0, keepdims=True)
    upd = jnp.concatenate([su, sq, jnp.zeros((6, D), jnp.float32)], axis=0)

    @pl.when(i == 0)
    def _():
        st_ref[...] = jnp.zeros((8, D), jnp.float32)

    st_ref[...] += upd


def _norm_body(agg_ref, st_ref, prm_ref, x_ref, out_ref):
    mean = st_ref[0:1, :] / float(N)
    ex2 = st_ref[1:2, :] / float(N)
    var = ex2 - mean * mean
    rstd = lax.rsqrt(var + EPS)
    g = prm_ref[1:2, :]
    be = prm_ref[2:3, :]
    h = (agg_ref[...] - mean) * rstd * g + be
    out_ref[...] = jnp.maximum(h, 0.0) + x_ref[...]


def kernel(x, edge_index, W, b, gamma, beta):
    ei = edge_index.astype(jnp.int32)
    pad_idx = (jnp.arange(E_PAD - E, dtype=jnp.int32) % N_PAD_ROWS) + N
    src2d = jnp.concatenate([ei[0], pad_idx]).reshape(E_PAD // CHUNK, CHUNK)
    dst2d = jnp.concatenate([ei[1], pad_idx]).reshape(E_PAD // CHUNK, CHUNK)
    x_ext = jnp.zeros((N_EXT, D), jnp.float32).at[:N].set(x)
    prm = jnp.zeros((8, D), jnp.float32).at[0].set(b).at[1].set(gamma).at[2].set(beta)

    degp = _deg_kernel(dst2d)
    deg0 = degp[0].reshape(N_EXT, 1)
    deg1 = degp[1].reshape(N_EXT, 1)

    y_ext, dinv = pl.pallas_call(
        _prep_body,
        grid=(N_EXT // RB,),
        in_specs=[
            pl.BlockSpec((RB, D), lambda i: (i, 0)),
            pl.BlockSpec((D, D), lambda i: (0, 0)),
            pl.BlockSpec((RB, 1), lambda i: (i, 0)),
            pl.BlockSpec((RB, 1), lambda i: (i, 0)),
        ],
        out_specs=[
            pl.BlockSpec((RB, D), lambda i: (i, 0)),
            pl.BlockSpec((RB, 1), lambda i: (i, 0)),
        ],
        out_shape=[
            jax.ShapeDtypeStruct((N_EXT, D), jnp.float32),
            jax.ShapeDtypeStruct((N_EXT, 1), jnp.float32),
        ],
    )(x_ext, W, deg0, deg1)

    acc = _scatter_kernel(src2d, dst2d, y_ext)

    agg, stats = pl.pallas_call(
        _stats_body,
        grid=(N // RBN,),
        in_specs=[
            pl.BlockSpec((1, RBN, D), lambda i: (0, i, 0)),
            pl.BlockSpec((1, RBN, D), lambda i: (1, i, 0)),
            pl.BlockSpec((RBN, D), lambda i: (i, 0)),
            pl.BlockSpec((RBN, 1), lambda i: (i, 0)),
            pl.BlockSpec((8, D), lambda i: (0, 0)),
        ],
        out_specs=[
            pl.BlockSpec((RBN, D), lambda i: (i, 0)),
            pl.BlockSpec((8, D), lambda i: (0, 0)),
        ],
        out_shape=[
            jax.ShapeDtypeStruct((N, D), jnp.float32),
            jax.ShapeDtypeStruct((8, D), jnp.float32),
        ],
    )(acc, acc, y_ext, dinv, prm)

    out = pl.pallas_call(
        _norm_body,
        grid=(N // RBN,),
        in_specs=[
            pl.BlockSpec((RBN, D), lambda i: (i, 0)),
            pl.BlockSpec((8, D), lambda i: (0, 0)),
            pl.BlockSpec((8, D), lambda i: (0, 0)),
            pl.BlockSpec((RBN, D), lambda i: (i, 0)),
        ],
        out_specs=pl.BlockSpec((RBN, D), lambda i: (i, 0)),
        out_shape=jax.ShapeDtypeStruct((N, D), jnp.float32),
    )(agg, stats, prm, x)

    return out


# speedup vs baseline: 36.4515x; 1.4485x over previous
import functools

import jax
import jax.numpy as jnp
from jax import lax
from jax.experimental import pallas as pl
from jax.experimental.pallas import tpu as pltpu
from jax.experimental.pallas import tpu_sc as plsc

N = 10000
D = 128
E = 320000
EPS = 1e-5

NC, NS = 2, 16
NW = NC * NS
N_EXT = 10240
N_PAD_ROWS = 64
E_PAD = 327680
CHUNK = 128
EPT_ROWS = E_PAD // NW // CHUNK
HROWS = EPT_ROWS // 2
RPT = N_EXT // NS
NBUF = 2

_mesh = lambda: plsc.VectorSubcoreMesh(core_axis_name="c", subcore_axis_name="s")


DSEM = 4


@functools.partial(
    pl.kernel,
    out_type=jax.ShapeDtypeStruct((NC, N_EXT), jnp.float32),
    mesh=_mesh(),
    scratch_types=[
        pltpu.VMEM((EPT_ROWS, CHUNK), jnp.int32),
        pltpu.VMEM((CHUNK,), jnp.float32),
        pltpu.VMEM((RPT,), jnp.float32),
        [pltpu.SemaphoreType.DMA for _ in range(DSEM)],
        pltpu.VMEM_SHARED((N_EXT,), jnp.float32),
    ],
)
def _deg_kernel(dst_hbm, degp_hbm, dst_v, ones_v, z_v, sems, deg_sh):
    cid = lax.axis_index("c")
    sid = lax.axis_index("s")
    wid = sid * NC + cid
    for i in range(CHUNK // 16):
        ones_v[pl.ds(i * 16, 16)] = jnp.ones((16,), jnp.float32)
    for i in range(RPT // 16):
        z_v[pl.ds(i * 16, 16)] = jnp.zeros((16,), jnp.float32)
    pltpu.sync_copy(dst_hbm.at[pl.ds(wid * EPT_ROWS, EPT_ROWS)], dst_v)
    pltpu.sync_copy(z_v, deg_sh.at[pl.ds(sid * RPT, RPT)])
    plsc.subcore_barrier()

    def scat(j, s):
        pltpu.async_copy(ones_v, deg_sh.at[dst_v.at[j]], sems[s], add=True)

    def dwait(s):
        pltpu.make_async_copy(ones_v, deg_sh.at[dst_v.at[0]], sems[s]).wait()

    for s in range(DSEM):
        scat(s, s)

    def body(o, carry):
        for s in range(DSEM):
            dwait(s)

            @pl.when(o < EPT_ROWS // DSEM - 1)
            def _():
                scat((o + 1) * DSEM + s, s)

        return carry

    lax.fori_loop(0, EPT_ROWS // DSEM, body, 0)
    plsc.subcore_barrier()
    pltpu.sync_copy(deg_sh.at[pl.ds(sid * RPT, RPT)],
                    degp_hbm.at[cid, pl.ds(sid * RPT, RPT)])


@functools.partial(
    pl.kernel,
    out_type=jax.ShapeDtypeStruct((NC, N_EXT, D), jnp.float32),
    mesh=_mesh(),
    scratch_types=[
        pltpu.VMEM((HROWS, CHUNK), jnp.int32),
        pltpu.VMEM((HROWS, CHUNK), jnp.int32),
        [pltpu.VMEM((CHUNK, D), jnp.float32) for _ in range(NBUF)],
        [pltpu.SemaphoreType.DMA for _ in range(NBUF)],
        [pltpu.SemaphoreType.DMA for _ in range(NBUF)],
        pltpu.VMEM_SHARED((N_EXT, D), jnp.float32),
    ],
)
def _scatter_kernel(src_hbm, dst_hbm, y_hbm, acc_hbm, src_v, dst_v, bufs, gsems,
                    ssems, acc_sh):
    cid = lax.axis_index("c")
    sid = lax.axis_index("s")
    wid = sid * NC + cid

    def gather(j, b):
        pltpu.async_copy(y_hbm.at[src_v.at[j]], bufs[b], gsems[b])

    def gwait(b):
        pltpu.make_async_copy(y_hbm.at[src_v.at[0]], bufs[b], gsems[b]).wait()

    def scatter(j, b):
        pltpu.async_copy(bufs[b], acc_sh.at[dst_v.at[j]], ssems[b], add=True)

    def swait(b):
        pltpu.make_async_copy(bufs[b], acc_sh.at[dst_v.at[0]], ssems[b]).wait()

    def ring():
        for b in range(NBUF):
            gather(b, b)

        def body(o, carry):
            for b in range(NBUF):
                j = o * NBUF + b
                gwait(b)
                scatter(j, b)
                swait(b)

                @pl.when(j + NBUF < HROWS)
                def _():
                    gather(j + NBUF, b)

            return carry

        lax.fori_loop(0, HROWS // NBUF, body, 0)

    pltpu.sync_copy(src_hbm.at[pl.ds(wid * EPT_ROWS, HROWS)], src_v)
    pltpu.sync_copy(dst_hbm.at[pl.ds(wid * EPT_ROWS, HROWS)], dst_v)
    pltpu.sync_copy(y_hbm.at[pl.ds(sid * RPT, RPT)],
                    acc_sh.at[pl.ds(sid * RPT, RPT)])
    plsc.subcore_barrier()
    ring()
    pltpu.sync_copy(src_hbm.at[pl.ds(wid * EPT_ROWS + HROWS, HROWS)], src_v)
    pltpu.sync_copy(dst_hbm.at[pl.ds(wid * EPT_ROWS + HROWS, HROWS)], dst_v)
    ring()
    plsc.subcore_barrier()
    pltpu.sync_copy(acc_sh.at[pl.ds(sid * RPT, RPT)],
                    acc_hbm.at[cid, pl.ds(sid * RPT, RPT)])


RB = 1024
RBN = 1000


def _prep_body(x_ref, w_ref, d0_ref, d1_ref, y_ref, dinv_ref):
    deg = d0_ref[...] + d1_ref[...] + 1.0
    dinv = lax.rsqrt(deg)
    xw = jnp.dot(x_ref[...], w_ref[...], preferred_element_type=jnp.float32)
    y_ref[...] = xw * dinv
    dinv_ref[...] = dinv


def _stats_body(p0_ref, p1_ref, y_ref, dinv_ref, prm_ref, agg_ref, st_ref):
    i = pl.program_id(0)
    s = p0_ref[0] + p1_ref[0] - y_ref[...]
    b = prm_ref[0:1, :]
    agg = s * dinv_ref[...] + b
    agg_ref[...] = agg
    su = jnp.sum(agg, axis=0, keepdims=True)
    sq = jnp.sum(agg * agg, axis=0, keepdims=True)
    upd = jnp.concatenate([su, sq, jnp.zeros((6, D), jnp.float32)], axis=0)

    @pl.when(i == 0)
    def _():
        st_ref[...] = jnp.zeros((8, D), jnp.float32)

    st_ref[...] += upd


def _norm_body(agg_ref, st_ref, prm_ref, x_ref, out_ref):
    mean = st_ref[0:1, :] / float(N)
    ex2 = st_ref[1:2, :] / float(N)
    var = ex2 - mean * mean
    rstd = lax.rsqrt(var + EPS)
    g = prm_ref[1:2, :]
    be = prm_ref[2:3, :]
    h = (agg_ref[...] - mean) * rstd * g + be
    out_ref[...] = jnp.maximum(h, 0.0) + x_ref[...]


def kernel(x, edge_index, W, b, gamma, beta):
    ei = edge_index.astype(jnp.int32)
    pad_idx = (jnp.arange(E_PAD - E, dtype=jnp.int32) % N_PAD_ROWS) + N
    src2d = jnp.concatenate([ei[0], pad_idx]).reshape(E_PAD // CHUNK, CHUNK)
    dst2d = jnp.concatenate([ei[1], pad_idx]).reshape(E_PAD // CHUNK, CHUNK)
    x_ext = jnp.zeros((N_EXT, D), jnp.float32).at[:N].set(x)
    prm = jnp.zeros((8, D), jnp.float32).at[0].set(b).at[1].set(gamma).at[2].set(beta)

    degp = _deg_kernel(dst2d)
    deg0 = degp[0].reshape(N_EXT, 1)
    deg1 = degp[1].reshape(N_EXT, 1)

    y_ext, dinv = pl.pallas_call(
        _prep_body,
        grid=(N_EXT // RB,),
        in_specs=[
            pl.BlockSpec((RB, D), lambda i: (i, 0)),
            pl.BlockSpec((D, D), lambda i: (0, 0)),
            pl.BlockSpec((RB, 1), lambda i: (i, 0)),
            pl.BlockSpec((RB, 1), lambda i: (i, 0)),
        ],
        out_specs=[
            pl.BlockSpec((RB, D), lambda i: (i, 0)),
            pl.BlockSpec((RB, 1), lambda i: (i, 0)),
        ],
        out_shape=[
            jax.ShapeDtypeStruct((N_EXT, D), jnp.float32),
            jax.ShapeDtypeStruct((N_EXT, 1), jnp.float32),
        ],
    )(x_ext, W, deg0, deg1)

    acc = _scatter_kernel(src2d, dst2d, y_ext)

    agg, stats = pl.pallas_call(
        _stats_body,
        grid=(N // RBN,),
        in_specs=[
            pl.BlockSpec((1, RBN, D), lambda i: (0, i, 0)),
            pl.BlockSpec((1, RBN, D), lambda i: (1, i, 0)),
            pl.BlockSpec((RBN, D), lambda i: (i, 0)),
            pl.BlockSpec((RBN, 1), lambda i: (i, 0)),
            pl.BlockSpec((8, D), lambda i: (0, 0)),
        ],
        out_specs=[
            pl.BlockSpec((RBN, D), lambda i: (i, 0)),
            pl.BlockSpec((8, D), lambda i: (0, 0)),
        ],
        out_shape=[
            jax.ShapeDtypeStruct((N, D), jnp.float32),
            jax.ShapeDtypeStruct((8, D), jnp.float32),
        ],
    )(acc, acc, y_ext, dinv, prm)

    out = pl.pallas_call(
        _norm_body,
        grid=(N // RBN,),
        in_specs=[
            pl.BlockSpec((RBN, D), lambda i: (i, 0)),
            pl.BlockSpec((8, D), lambda i: (0, 0)),
            pl.BlockSpec((8, D), lambda i: (0, 0)),
            pl.BlockSpec((RBN, D), lambda i: (i, 0)),
        ],
        out_specs=pl.BlockSpec((RBN, D), lambda i: (i, 0)),
        out_shape=jax.ShapeDtypeStruct((N, D), jnp.float32),
    )(agg, stats, prm, x)

    return out


# speedup vs baseline: 36.6129x; 1.0044x over previous
import functools

import jax
import jax.numpy as jnp
from jax import lax
from jax.experimental import pallas as pl
from jax.experimental.pallas import tpu as pltpu
from jax.experimental.pallas import tpu_sc as plsc

N = 10000
D = 128
E = 320000
EPS = 1e-5

NC, NS = 2, 16
NW = NC * NS
N_EXT = 10240
N_PAD_ROWS = 64
E_PAD = 327680
CHUNK = 128
EPT_ROWS = E_PAD // NW // CHUNK
HROWS = EPT_ROWS // 2
RPT = N_EXT // NS
IPT8 = (N // NS) // 8 * 8
NBUF = 2

_mesh = lambda: plsc.VectorSubcoreMesh(core_axis_name="c", subcore_axis_name="s")


DSEM = 4


@functools.partial(
    pl.kernel,
    out_type=jax.ShapeDtypeStruct((NC, N_EXT), jnp.float32),
    mesh=_mesh(),
    scratch_types=[
        pltpu.VMEM((EPT_ROWS, CHUNK), jnp.int32),
        pltpu.VMEM((CHUNK,), jnp.float32),
        pltpu.VMEM((RPT,), jnp.float32),
        [pltpu.SemaphoreType.DMA for _ in range(DSEM)],
        pltpu.VMEM_SHARED((N_EXT,), jnp.float32),
    ],
)
def _deg_kernel(dst_hbm, degp_hbm, dst_v, ones_v, z_v, sems, deg_sh):
    cid = lax.axis_index("c")
    sid = lax.axis_index("s")
    wid = sid * NC + cid
    for i in range(CHUNK // 16):
        ones_v[pl.ds(i * 16, 16)] = jnp.ones((16,), jnp.float32)
    for i in range(RPT // 16):
        z_v[pl.ds(i * 16, 16)] = jnp.zeros((16,), jnp.float32)
    pltpu.sync_copy(dst_hbm.at[pl.ds(wid * EPT_ROWS, EPT_ROWS)], dst_v)
    pltpu.sync_copy(z_v, deg_sh.at[pl.ds(sid * RPT, RPT)])
    plsc.subcore_barrier()

    def scat(j, s):
        pltpu.async_copy(ones_v, deg_sh.at[dst_v.at[j]], sems[s], add=True)

    def dwait(s):
        pltpu.make_async_copy(ones_v, deg_sh.at[dst_v.at[0]], sems[s]).wait()

    for s in range(DSEM):
        scat(s, s)

    def body(o, carry):
        for s in range(DSEM):
            dwait(s)

            @pl.when(o < EPT_ROWS // DSEM - 1)
            def _():
                scat((o + 1) * DSEM + s, s)

        return carry

    lax.fori_loop(0, EPT_ROWS // DSEM, body, 0)
    plsc.subcore_barrier()
    pltpu.sync_copy(deg_sh.at[pl.ds(sid * RPT, RPT)],
                    degp_hbm.at[cid, pl.ds(sid * RPT, RPT)])


@functools.partial(
    pl.kernel,
    out_type=jax.ShapeDtypeStruct((NC, N_EXT, D), jnp.float32),
    mesh=_mesh(),
    scratch_types=[
        pltpu.VMEM((HROWS, CHUNK), jnp.int32),
        pltpu.VMEM((HROWS, CHUNK), jnp.int32),
        [pltpu.VMEM((CHUNK, D), jnp.float32) for _ in range(NBUF)],
        [pltpu.SemaphoreType.DMA for _ in range(NBUF)],
        [pltpu.SemaphoreType.DMA for _ in range(NBUF)],
        pltpu.VMEM_SHARED((N_EXT, D), jnp.float32),
    ],
)
def _scatter_kernel(src_hbm, dst_hbm, y_hbm, acc_hbm, src_v, dst_v, bufs, gsems,
                    ssems, acc_sh):
    cid = lax.axis_index("c")
    sid = lax.axis_index("s")
    wid = sid * NC + cid

    def gather(j, b):
        pltpu.async_copy(y_hbm.at[src_v.at[j]], bufs[b], gsems[b])

    def gwait(b):
        pltpu.make_async_copy(y_hbm.at[src_v.at[0]], bufs[b], gsems[b]).wait()

    def scatter(j, b):
        pltpu.async_copy(bufs[b], acc_sh.at[dst_v.at[j]], ssems[b], add=True)

    def swait(b):
        pltpu.make_async_copy(bufs[b], acc_sh.at[dst_v.at[0]], ssems[b]).wait()

    def ring():
        for b in range(NBUF):
            gather(b, b)

        def body(o, carry):
            for b in range(NBUF):
                j = o * NBUF + b
                gwait(b)
                scatter(j, b)
                swait(b)

                @pl.when(j + NBUF < HROWS)
                def _():
                    gather(j + NBUF, b)

            return carry

        lax.fori_loop(0, HROWS // NBUF, body, 0)

    pltpu.sync_copy(src_hbm.at[pl.ds(wid * EPT_ROWS, HROWS)], src_v)
    pltpu.sync_copy(dst_hbm.at[pl.ds(wid * EPT_ROWS, HROWS)], dst_v)
    pltpu.sync_copy(y_hbm.at[pl.ds(sid * IPT8, IPT8)],
                    acc_sh.at[pl.ds(sid * IPT8, IPT8)])

    @pl.when(sid == NS - 1)
    def _():
        pltpu.sync_copy(y_hbm.at[pl.ds(NS * IPT8, N - NS * IPT8)],
                        acc_sh.at[pl.ds(NS * IPT8, N - NS * IPT8)])
    plsc.subcore_barrier()
    ring()
    pltpu.sync_copy(src_hbm.at[pl.ds(wid * EPT_ROWS + HROWS, HROWS)], src_v)
    pltpu.sync_copy(dst_hbm.at[pl.ds(wid * EPT_ROWS + HROWS, HROWS)], dst_v)
    ring()
    plsc.subcore_barrier()
    pltpu.sync_copy(acc_sh.at[pl.ds(sid * RPT, RPT)],
                    acc_hbm.at[cid, pl.ds(sid * RPT, RPT)])


RB = 1024
RBN = 1000


def _prep_body(x_ref, w_ref, d0_ref, d1_ref, y_ref, dinv_ref):
    deg = d0_ref[...] + d1_ref[...] + 1.0
    dinv = lax.rsqrt(deg)
    xw = jnp.dot(x_ref[...], w_ref[...], preferred_element_type=jnp.float32)
    y_ref[...] = xw * dinv
    dinv_ref[...] = dinv


def _stats_body(p0_ref, p1_ref, y_ref, dinv_ref, prm_ref, agg_ref, st_ref):
    i = pl.program_id(0)
    s = p0_ref[0] + p1_ref[0] - y_ref[...]
    b = prm_ref[0:1, :]
    agg = s * dinv_ref[...] + b
    agg_ref[...] = agg
    su = jnp.sum(agg, axis=0, keepdims=True)
    sq = jnp.sum(agg * agg, axis=0, keepdims=True)
    upd = jnp.concatenate([su, sq, jnp.zeros((6, D), jnp.float32)], axis=0)

    @pl.when(i == 0)
    def _():
        st_ref[...] = jnp.zeros((8, D), jnp.float32)

    st_ref[...] += upd


def _norm_body(agg_ref, st_ref, prm_ref, x_ref, out_ref):
    mean = st_ref[0:1, :] / float(N)
    ex2 = st_ref[1:2, :] / float(N)
    var = ex2 - mean * mean
    rstd = lax.rsqrt(var + EPS)
    g = prm_ref[1:2, :]
    be = prm_ref[2:3, :]
    h = (agg_ref[...] - mean) * rstd * g + be
    out_ref[...] = jnp.maximum(h, 0.0) + x_ref[...]


def kernel(x, edge_index, W, b, gamma, beta):
    ei = edge_index.astype(jnp.int32)
    npad = E_PAD - E
    pad_src = jnp.arange(npad, dtype=jnp.int32) % N_PAD_ROWS
    pad_dst = pad_src + N
    src2d = jnp.concatenate([ei[0], pad_src]).reshape(E_PAD // CHUNK, CHUNK)
    dst2d = jnp.concatenate([ei[1], pad_dst]).reshape(E_PAD // CHUNK, CHUNK)
    prm = jnp.zeros((8, D), jnp.float32).at[0].set(b).at[1].set(gamma).at[2].set(beta)

    degp = _deg_kernel(dst2d)
    deg0 = degp[0, :N].reshape(N, 1)
    deg1 = degp[1, :N].reshape(N, 1)

    y, dinv = pl.pallas_call(
        _prep_body,
        grid=(N // RBN,),
        in_specs=[
            pl.BlockSpec((RBN, D), lambda i: (i, 0)),
            pl.BlockSpec((D, D), lambda i: (0, 0)),
            pl.BlockSpec((RBN, 1), lambda i: (i, 0)),
            pl.BlockSpec((RBN, 1), lambda i: (i, 0)),
        ],
        out_specs=[
            pl.BlockSpec((RBN, D), lambda i: (i, 0)),
            pl.BlockSpec((RBN, 1), lambda i: (i, 0)),
        ],
        out_shape=[
            jax.ShapeDtypeStruct((N, D), jnp.float32),
            jax.ShapeDtypeStruct((N, 1), jnp.float32),
        ],
    )(x, W, deg0, deg1)

    acc = _scatter_kernel(src2d, dst2d, y)

    agg, stats = pl.pallas_call(
        _stats_body,
        grid=(N // RBN,),
        in_specs=[
            pl.BlockSpec((1, RBN, D), lambda i: (0, i, 0)),
            pl.BlockSpec((1, RBN, D), lambda i: (1, i, 0)),
            pl.BlockSpec((RBN, D), lambda i: (i, 0)),
            pl.BlockSpec((RBN, 1), lambda i: (i, 0)),
            pl.BlockSpec((8, D), lambda i: (0, 0)),
        ],
        out_specs=[
            pl.BlockSpec((RBN, D), lambda i: (i, 0)),
            pl.BlockSpec((8, D), lambda i: (0, 0)),
        ],
        out_shape=[
            jax.ShapeDtypeStruct((N, D), jnp.float32),
            jax.ShapeDtypeStruct((8, D), jnp.float32),
        ],
    )(acc, acc, y, dinv, prm)

    out = pl.pallas_call(
        _norm_body,
        grid=(N // RBN,),
        in_specs=[
            pl.BlockSpec((RBN, D), lambda i: (i, 0)),
            pl.BlockSpec((8, D), lambda i: (0, 0)),
            pl.BlockSpec((8, D), lambda i: (0, 0)),
            pl.BlockSpec((RBN, D), lambda i: (i, 0)),
        ],
        out_specs=pl.BlockSpec((RBN, D), lambda i: (i, 0)),
        out_shape=jax.ShapeDtypeStruct((N, D), jnp.float32),
    )(agg, stats, prm, x)

    return out


# speedup vs baseline: 37.7887x; 1.0321x over previous
import functools

import jax
import jax.numpy as jnp
from jax import lax
from jax.experimental import pallas as pl
from jax.experimental.pallas import tpu as pltpu
from jax.experimental.pallas import tpu_sc as plsc

N = 10000
D = 128
E = 320000
EPS = 1e-5

NC, NS = 2, 16
NW = NC * NS
N_EXT = 10240
N_PAD_ROWS = 64
E_PAD = 327680
CHUNK = 64
SC_ROWS = E_PAD // NW // CHUNK
QROWS = SC_ROWS // 4
DEG_CHUNK = 128
EPT_ROWS = E_PAD // NW // DEG_CHUNK
RPT = N_EXT // NS
IPT8 = (N // NS) // 8 * 8
NBUF = 4

_mesh = lambda: plsc.VectorSubcoreMesh(core_axis_name="c", subcore_axis_name="s")


DSEM = 4


@functools.partial(
    pl.kernel,
    out_type=jax.ShapeDtypeStruct((NC, N_EXT), jnp.float32),
    mesh=_mesh(),
    scratch_types=[
        pltpu.VMEM((EPT_ROWS, DEG_CHUNK), jnp.int32),
        pltpu.VMEM((DEG_CHUNK,), jnp.float32),
        pltpu.VMEM((RPT,), jnp.float32),
        [pltpu.SemaphoreType.DMA for _ in range(DSEM)],
        pltpu.VMEM_SHARED((N_EXT,), jnp.float32),
    ],
)
def _deg_kernel(dst_hbm, degp_hbm, dst_v, ones_v, z_v, sems, deg_sh):
    cid = lax.axis_index("c")
    sid = lax.axis_index("s")
    wid = sid * NC + cid
    for i in range(DEG_CHUNK // 16):
        ones_v[pl.ds(i * 16, 16)] = jnp.ones((16,), jnp.float32)
    for i in range(RPT // 16):
        z_v[pl.ds(i * 16, 16)] = jnp.zeros((16,), jnp.float32)
    pltpu.sync_copy(dst_hbm.at[pl.ds(wid * EPT_ROWS, EPT_ROWS)], dst_v)
    pltpu.sync_copy(z_v, deg_sh.at[pl.ds(sid * RPT, RPT)])
    plsc.subcore_barrier()

    def scat(j, s):
        pltpu.async_copy(ones_v, deg_sh.at[dst_v.at[j]], sems[s], add=True)

    def dwait(s):
        pltpu.make_async_copy(ones_v, deg_sh.at[dst_v.at[0]], sems[s]).wait()

    for s in range(DSEM):
        scat(s, s)

    def body(o, carry):
        for s in range(DSEM):
            dwait(s)

            @pl.when(o < EPT_ROWS // DSEM - 1)
            def _():
                scat((o + 1) * DSEM + s, s)

        return carry

    lax.fori_loop(0, EPT_ROWS // DSEM, body, 0)
    plsc.subcore_barrier()
    pltpu.sync_copy(deg_sh.at[pl.ds(sid * RPT, RPT)],
                    degp_hbm.at[cid, pl.ds(sid * RPT, RPT)])


@functools.partial(
    pl.kernel,
    out_type=jax.ShapeDtypeStruct((NC, N_EXT, D), jnp.float32),
    mesh=_mesh(),
    scratch_types=[
        pltpu.VMEM((QROWS, CHUNK), jnp.int32),
        pltpu.VMEM((QROWS, CHUNK), jnp.int32),
        [pltpu.VMEM((CHUNK, D), jnp.float32) for _ in range(NBUF)],
        [pltpu.SemaphoreType.DMA for _ in range(NBUF)],
        [pltpu.SemaphoreType.DMA for _ in range(NBUF)],
        pltpu.VMEM_SHARED((N_EXT, D), jnp.float32),
    ],
)
def _scatter_kernel(src_hbm, dst_hbm, y_hbm, acc_hbm, src_v, dst_v, bufs, gsems,
                    ssems, acc_sh):
    cid = lax.axis_index("c")
    sid = lax.axis_index("s")
    wid = sid * NC + cid

    def gather(j, b):
        pltpu.async_copy(y_hbm.at[src_v.at[j]], bufs[b], gsems[b])

    def gwait(b):
        pltpu.make_async_copy(y_hbm.at[src_v.at[0]], bufs[b], gsems[b]).wait()

    def scatter(j, b):
        pltpu.async_copy(bufs[b], acc_sh.at[dst_v.at[j]], ssems[b], add=True)

    def swait(b):
        pltpu.make_async_copy(bufs[b], acc_sh.at[dst_v.at[0]], ssems[b]).wait()

    def ring():
        for b in range(NBUF):
            gather(b, b)

        def body(o, carry):
            for b in range(NBUF):
                j = o * NBUF + b
                gwait(b)
                scatter(j, b)
                swait(b)

                @pl.when(j + NBUF < QROWS)
                def _():
                    gather(j + NBUF, b)

            return carry

        lax.fori_loop(0, QROWS // NBUF, body, 0)

    pltpu.sync_copy(src_hbm.at[pl.ds(wid * SC_ROWS, QROWS)], src_v)
    pltpu.sync_copy(dst_hbm.at[pl.ds(wid * SC_ROWS, QROWS)], dst_v)
    pltpu.sync_copy(y_hbm.at[pl.ds(sid * IPT8, IPT8)],
                    acc_sh.at[pl.ds(sid * IPT8, IPT8)])

    @pl.when(sid == NS - 1)
    def _():
        pltpu.sync_copy(y_hbm.at[pl.ds(NS * IPT8, N - NS * IPT8)],
                        acc_sh.at[pl.ds(NS * IPT8, N - NS * IPT8)])
    plsc.subcore_barrier()
    ring()
    for seg in range(1, 4):
        pltpu.sync_copy(src_hbm.at[pl.ds(wid * SC_ROWS + seg * QROWS, QROWS)], src_v)
        pltpu.sync_copy(dst_hbm.at[pl.ds(wid * SC_ROWS + seg * QROWS, QROWS)], dst_v)
        ring()
    plsc.subcore_barrier()
    pltpu.sync_copy(acc_sh.at[pl.ds(sid * RPT, RPT)],
                    acc_hbm.at[cid, pl.ds(sid * RPT, RPT)])


RB = 1024
RBN = 1000


def _prep_body(x_ref, w_ref, d0_ref, d1_ref, y_ref, dinv_ref):
    deg = d0_ref[...] + d1_ref[...] + 1.0
    dinv = lax.rsqrt(deg)
    xw = jnp.dot(x_ref[...], w_ref[...], preferred_element_type=jnp.float32)
    y_ref[...] = xw * dinv
    dinv_ref[...] = dinv


def _stats_body(p0_ref, p1_ref, y_ref, dinv_ref, prm_ref, agg_ref, st_ref):
    i = pl.program_id(0)
    s = p0_ref[0] + p1_ref[0] - y_ref[...]
    b = prm_ref[0:1, :]
    agg = s * dinv_ref[...] + b
    agg_ref[...] = agg
    su = jnp.sum(agg, axis=0, keepdims=True)
    sq = jnp.sum(agg * agg, axis=0, keepdims=True)
    upd = jnp.concatenate([su, sq, jnp.zeros((6, D), jnp.float32)], axis=0)

    @pl.when(i == 0)
    def _():
        st_ref[...] = jnp.zeros((8, D), jnp.float32)

    st_ref[...] += upd


def _norm_body(agg_ref, st_ref, prm_ref, x_ref, out_ref):
    mean = st_ref[0:1, :] / float(N)
    ex2 = st_ref[1:2, :] / float(N)
    var = ex2 - mean * mean
    rstd = lax.rsqrt(var + EPS)
    g = prm_ref[1:2, :]
    be = prm_ref[2:3, :]
    h = (agg_ref[...] - mean) * rstd * g + be
    out_ref[...] = jnp.maximum(h, 0.0) + x_ref[...]


def kernel(x, edge_index, W, b, gamma, beta):
    ei = edge_index.astype(jnp.int32)
    npad = E_PAD - E
    pad_src = jnp.arange(npad, dtype=jnp.int32) % N_PAD_ROWS
    pad_dst = pad_src + N
    src_flat = jnp.concatenate([ei[0], pad_src])
    dst_flat = jnp.concatenate([ei[1], pad_dst])
    src2d = src_flat.reshape(E_PAD // CHUNK, CHUNK)
    dst2d = dst_flat.reshape(E_PAD // CHUNK, CHUNK)
    dst2d_deg = dst_flat.reshape(E_PAD // DEG_CHUNK, DEG_CHUNK)
    prm = jnp.zeros((8, D), jnp.float32).at[0].set(b).at[1].set(gamma).at[2].set(beta)

    degp = _deg_kernel(dst2d_deg)
    deg0 = degp[0, :N].reshape(N, 1)
    deg1 = degp[1, :N].reshape(N, 1)

    y, dinv = pl.pallas_call(
        _prep_body,
        grid=(N // RBN,),
        in_specs=[
            pl.BlockSpec((RBN, D), lambda i: (i, 0)),
            pl.BlockSpec((D, D), lambda i: (0, 0)),
            pl.BlockSpec((RBN, 1), lambda i: (i, 0)),
            pl.BlockSpec((RBN, 1), lambda i: (i, 0)),
        ],
        out_specs=[
            pl.BlockSpec((RBN, D), lambda i: (i, 0)),
            pl.BlockSpec((RBN, 1), lambda i: (i, 0)),
        ],
        out_shape=[
            jax.ShapeDtypeStruct((N, D), jnp.float32),
            jax.ShapeDtypeStruct((N, 1), jnp.float32),
        ],
    )(x, W, deg0, deg1)

    acc = _scatter_kernel(src2d, dst2d, y)

    agg, stats = pl.pallas_call(
        _stats_body,
        grid=(N // RBN,),
        in_specs=[
            pl.BlockSpec((1, RBN, D), lambda i: (0, i, 0)),
            pl.BlockSpec((1, RBN, D), lambda i: (1, i, 0)),
            pl.BlockSpec((RBN, D), lambda i: (i, 0)),
            pl.BlockSpec((RBN, 1), lambda i: (i, 0)),
            pl.BlockSpec((8, D), lambda i: (0, 0)),
        ],
        out_specs=[
            pl.BlockSpec((RBN, D), lambda i: (i, 0)),
            pl.BlockSpec((8, D), lambda i: (0, 0)),
        ],
        out_shape=[
            jax.ShapeDtypeStruct((N, D), jnp.float32),
            jax.ShapeDtypeStruct((8, D), jnp.float32),
        ],
    )(acc, acc, y, dinv, prm)

    out = pl.pallas_call(
        _norm_body,
        grid=(N // RBN,),
        in_specs=[
            pl.BlockSpec((RBN, D), lambda i: (i, 0)),
            pl.BlockSpec((8, D), lambda i: (0, 0)),
            pl.BlockSpec((8, D), lambda i: (0, 0)),
            pl.BlockSpec((RBN, D), lambda i: (i, 0)),
        ],
        out_specs=pl.BlockSpec((RBN, D), lambda i: (i, 0)),
        out_shape=jax.ShapeDtypeStruct((N, D), jnp.float32),
    )(agg, stats, prm, x)

    return out


# speedup vs baseline: 38.2440x; 1.0121x over previous
import functools

import jax
import jax.numpy as jnp
from jax import lax
from jax.experimental import pallas as pl
from jax.experimental.pallas import tpu as pltpu
from jax.experimental.pallas import tpu_sc as plsc

N = 10000
D = 128
E = 320000
EPS = 1e-5

NC, NS = 2, 16
NW = NC * NS
N_EXT = 10240
N_PAD_ROWS = 64
E_PAD = 327680
CHUNK = 64
SC_ROWS = E_PAD // NW // CHUNK
QROWS = SC_ROWS // 4
DEG_CHUNK = 128
EPT_ROWS = E_PAD // NW // DEG_CHUNK
RPT = N_EXT // NS
IPT8 = (N // NS) // 8 * 8
NBUF = 4

_mesh = lambda: plsc.VectorSubcoreMesh(core_axis_name="c", subcore_axis_name="s")


DSEM = 4


@functools.partial(
    pl.kernel,
    out_type=jax.ShapeDtypeStruct((NC, N_EXT), jnp.float32),
    mesh=_mesh(),
    scratch_types=[
        pltpu.VMEM((EPT_ROWS, DEG_CHUNK), jnp.int32),
        pltpu.VMEM((DEG_CHUNK,), jnp.float32),
        pltpu.VMEM((RPT,), jnp.float32),
        [pltpu.SemaphoreType.DMA for _ in range(DSEM)],
        pltpu.VMEM_SHARED((N_EXT,), jnp.float32),
    ],
)
def _deg_kernel(dst_hbm, degp_hbm, dst_v, ones_v, z_v, sems, deg_sh):
    cid = lax.axis_index("c")
    sid = lax.axis_index("s")
    wid = sid * NC + cid
    for i in range(DEG_CHUNK // 16):
        ones_v[pl.ds(i * 16, 16)] = jnp.ones((16,), jnp.float32)
    for i in range(RPT // 16):
        z_v[pl.ds(i * 16, 16)] = jnp.zeros((16,), jnp.float32)
    pltpu.sync_copy(dst_hbm.at[pl.ds(wid * EPT_ROWS, EPT_ROWS)], dst_v)
    pltpu.sync_copy(z_v, deg_sh.at[pl.ds(sid * RPT, RPT)])
    plsc.subcore_barrier()

    def scat(j, s):
        pltpu.async_copy(ones_v, deg_sh.at[dst_v.at[j]], sems[s], add=True)

    def dwait(s):
        pltpu.make_async_copy(ones_v, deg_sh.at[dst_v.at[0]], sems[s]).wait()

    for s in range(DSEM):
        scat(s, s)

    def body(o, carry):
        for s in range(DSEM):
            dwait(s)

            @pl.when(o < EPT_ROWS // DSEM - 1)
            def _():
                scat((o + 1) * DSEM + s, s)

        return carry

    lax.fori_loop(0, EPT_ROWS // DSEM, body, 0)
    plsc.subcore_barrier()
    pltpu.sync_copy(deg_sh.at[pl.ds(sid * RPT, RPT)],
                    degp_hbm.at[cid, pl.ds(sid * RPT, RPT)])


@functools.partial(
    pl.kernel,
    out_type=jax.ShapeDtypeStruct((NC, N_EXT, D), jnp.float32),
    mesh=_mesh(),
    scratch_types=[
        pltpu.VMEM((QROWS, CHUNK), jnp.int32),
        pltpu.VMEM((QROWS, CHUNK), jnp.int32),
        [pltpu.VMEM((CHUNK, D), jnp.float32) for _ in range(NBUF)],
        [pltpu.SemaphoreType.DMA for _ in range(NBUF)],
        [pltpu.SemaphoreType.DMA for _ in range(NBUF)],
        pltpu.VMEM_SHARED((N_EXT, D), jnp.float32),
    ],
)
def _scatter_kernel(src_hbm, dst_hbm, y_hbm, acc_hbm, src_v, dst_v, bufs, gsems,
                    ssems, acc_sh):
    cid = lax.axis_index("c")
    sid = lax.axis_index("s")
    wid = sid * NC + cid

    def gather(j, b):
        pltpu.async_copy(y_hbm.at[src_v.at[j]], bufs[b], gsems[b])

    def gwait(b):
        pltpu.make_async_copy(y_hbm.at[src_v.at[0]], bufs[b], gsems[b]).wait()

    def scatter(j, b):
        pltpu.async_copy(bufs[b], acc_sh.at[dst_v.at[j]], ssems[b], add=True)

    def swait(b):
        pltpu.make_async_copy(bufs[b], acc_sh.at[dst_v.at[0]], ssems[b]).wait()

    def ring():
        for b in range(NBUF):
            gather(b, b)

        def body(o, carry):
            for b in range(NBUF):
                j = o * NBUF + b
                gwait(b)
                scatter(j, b)
                swait(b)

                @pl.when(j + NBUF < QROWS)
                def _():
                    gather(j + NBUF, b)

            return carry

        lax.fori_loop(0, QROWS // NBUF, body, 0)

    pltpu.sync_copy(src_hbm.at[pl.ds(wid * SC_ROWS, QROWS)], src_v)
    pltpu.sync_copy(dst_hbm.at[pl.ds(wid * SC_ROWS, QROWS)], dst_v)
    pltpu.sync_copy(y_hbm.at[pl.ds(sid * IPT8, IPT8)],
                    acc_sh.at[pl.ds(sid * IPT8, IPT8)])

    @pl.when(sid == NS - 1)
    def _():
        pltpu.sync_copy(y_hbm.at[pl.ds(NS * IPT8, N - NS * IPT8)],
                        acc_sh.at[pl.ds(NS * IPT8, N - NS * IPT8)])
    plsc.subcore_barrier()
    ring()
    for seg in range(1, 4):
        pltpu.sync_copy(src_hbm.at[pl.ds(wid * SC_ROWS + seg * QROWS, QROWS)], src_v)
        pltpu.sync_copy(dst_hbm.at[pl.ds(wid * SC_ROWS + seg * QROWS, QROWS)], dst_v)
        ring()
    plsc.subcore_barrier()
    pltpu.sync_copy(acc_sh.at[pl.ds(sid * RPT, RPT)],
                    acc_hbm.at[cid, pl.ds(sid * RPT, RPT)])


RB = 1024
RBN = 1000


def _prep_body(x_ref, w_ref, d0_ref, d1_ref, y_ref, dinv_ref):
    deg = d0_ref[...] + d1_ref[...] + 1.0
    dinv = lax.rsqrt(deg)
    xw = jnp.dot(x_ref[...], w_ref[...], preferred_element_type=jnp.float32)
    y_ref[...] = xw * dinv
    dinv_ref[...] = dinv


def _bn_body(p0_ref, p1_ref, y_ref, dinv_ref, prm_ref, x_ref, out_ref,
             agg_vmem, st_vmem):
    ph = pl.program_id(0)
    i = pl.program_id(1)

    @pl.when(ph == 0)
    def _():
        s = p0_ref[0] + p1_ref[0] - y_ref[...]
        agg = s * dinv_ref[...] + prm_ref[0:1, :]
        agg_vmem[pl.ds(i * RBN, RBN), :] = agg
        su = jnp.sum(agg, axis=0, keepdims=True)
        sq = jnp.sum(agg * agg, axis=0, keepdims=True)
        upd = jnp.concatenate([su, sq, jnp.zeros((6, D), jnp.float32)], axis=0)

        @pl.when(i == 0)
        def _():
            st_vmem[...] = jnp.zeros((8, D), jnp.float32)

        st_vmem[...] += upd

    @pl.when(ph == 1)
    def _():
        mean = st_vmem[0:1, :] / float(N)
        ex2 = st_vmem[1:2, :] / float(N)
        var = ex2 - mean * mean
        rstd = lax.rsqrt(var + EPS)
        g = prm_ref[1:2, :]
        be = prm_ref[2:3, :]
        agg = agg_vmem[pl.ds(i * RBN, RBN), :]
        h = (agg - mean) * rstd * g + be
        out_ref[...] = jnp.maximum(h, 0.0) + x_ref[...]


def _stats_body(p0_ref, p1_ref, y_ref, dinv_ref, prm_ref, agg_ref, st_ref):
    i = pl.program_id(0)
    s = p0_ref[0] + p1_ref[0] - y_ref[...]
    b = prm_ref[0:1, :]
    agg = s * dinv_ref[...] + b
    agg_ref[...] = agg
    su = jnp.sum(agg, axis=0, keepdims=True)
    sq = jnp.sum(agg * agg, axis=0, keepdims=True)
    upd = jnp.concatenate([su, sq, jnp.zeros((6, D), jnp.float32)], axis=0)

    @pl.when(i == 0)
    def _():
        st_ref[...] = jnp.zeros((8, D), jnp.float32)

    st_ref[...] += upd


def _norm_body(agg_ref, st_ref, prm_ref, x_ref, out_ref):
    mean = st_ref[0:1, :] / float(N)
    ex2 = st_ref[1:2, :] / float(N)
    var = ex2 - mean * mean
    rstd = lax.rsqrt(var + EPS)
    g = prm_ref[1:2, :]
    be = prm_ref[2:3, :]
    h = (agg_ref[...] - mean) * rstd * g + be
    out_ref[...] = jnp.maximum(h, 0.0) + x_ref[...]


def kernel(x, edge_index, W, b, gamma, beta):
    ei = edge_index.astype(jnp.int32)
    npad = E_PAD - E
    pad_src = jnp.arange(npad, dtype=jnp.int32) % N_PAD_ROWS
    pad_dst = pad_src + N
    src_flat = jnp.concatenate([ei[0], pad_src])
    dst_flat = jnp.concatenate([ei[1], pad_dst])
    src2d = src_flat.reshape(E_PAD // CHUNK, CHUNK)
    dst2d = dst_flat.reshape(E_PAD // CHUNK, CHUNK)
    dst2d_deg = dst_flat.reshape(E_PAD // DEG_CHUNK, DEG_CHUNK)
    prm = jnp.zeros((8, D), jnp.float32).at[0].set(b).at[1].set(gamma).at[2].set(beta)

    degp = _deg_kernel(dst2d_deg)
    deg0 = degp[0, :N].reshape(N, 1)
    deg1 = degp[1, :N].reshape(N, 1)

    y, dinv = pl.pallas_call(
        _prep_body,
        grid=(N // RBN,),
        in_specs=[
            pl.BlockSpec((RBN, D), lambda i: (i, 0)),
            pl.BlockSpec((D, D), lambda i: (0, 0)),
            pl.BlockSpec((RBN, 1), lambda i: (i, 0)),
            pl.BlockSpec((RBN, 1), lambda i: (i, 0)),
        ],
        out_specs=[
            pl.BlockSpec((RBN, D), lambda i: (i, 0)),
            pl.BlockSpec((RBN, 1), lambda i: (i, 0)),
        ],
        out_shape=[
            jax.ShapeDtypeStruct((N, D), jnp.float32),
            jax.ShapeDtypeStruct((N, 1), jnp.float32),
        ],
    )(x, W, deg0, deg1)

    acc = _scatter_kernel(src2d, dst2d, y)

    out = pl.pallas_call(
        _bn_body,
        grid=(2, N // RBN),
        in_specs=[
            pl.BlockSpec((1, RBN, D), lambda p, i: (0, i * (1 - p), 0)),
            pl.BlockSpec((1, RBN, D), lambda p, i: (1, i * (1 - p), 0)),
            pl.BlockSpec((RBN, D), lambda p, i: (i * (1 - p), 0)),
            pl.BlockSpec((RBN, 1), lambda p, i: (i * (1 - p), 0)),
            pl.BlockSpec((8, D), lambda p, i: (0, 0)),
            pl.BlockSpec((RBN, D), lambda p, i: (i * p, 0)),
        ],
        out_specs=pl.BlockSpec((RBN, D), lambda p, i: (i, 0)),
        out_shape=jax.ShapeDtypeStruct((N, D), jnp.float32),
        scratch_shapes=[
            pltpu.VMEM((N, D), jnp.float32),
            pltpu.VMEM((8, D), jnp.float32),
        ],
        compiler_params=pltpu.CompilerParams(
            dimension_semantics=("arbitrary", "arbitrary")),
    )(acc, acc, y, dinv, prm, x)

    return out


# speedup vs baseline: 38.7541x; 1.0133x over previous
import functools

import jax
import jax.numpy as jnp
from jax import lax
from jax.experimental import pallas as pl
from jax.experimental.pallas import tpu as pltpu
from jax.experimental.pallas import tpu_sc as plsc

N = 10000
D = 128
E = 320000
EPS = 1e-5

NC, NS = 2, 16
NW = NC * NS
N_EXT = 10240
N_PAD_ROWS = 64
E_PAD = 327680
CHUNK = 64
SC_ROWS = E_PAD // NW // CHUNK
QROWS = SC_ROWS // 4
DEG_ROWS = E_PAD // NW // CHUNK
RPT = N_EXT // NS
IPT8 = (N // NS) // 8 * 8
NBUF = 4

_mesh = lambda: plsc.VectorSubcoreMesh(core_axis_name="c", subcore_axis_name="s")


DSEM = 4


@functools.partial(
    pl.kernel,
    out_type=jax.ShapeDtypeStruct((NC, N_EXT), jnp.float32),
    mesh=_mesh(),
    scratch_types=[
        pltpu.VMEM((DEG_ROWS, CHUNK), jnp.int32),
        pltpu.VMEM((CHUNK,), jnp.float32),
        pltpu.VMEM((RPT,), jnp.float32),
        [pltpu.SemaphoreType.DMA for _ in range(DSEM)],
        pltpu.VMEM_SHARED((N_EXT,), jnp.float32),
    ],
)
def _deg_kernel(dst_hbm, degp_hbm, dst_v, ones_v, z_v, sems, deg_sh):
    cid = lax.axis_index("c")
    sid = lax.axis_index("s")
    wid = sid * NC + cid
    for i in range(CHUNK // 16):
        ones_v[pl.ds(i * 16, 16)] = jnp.ones((16,), jnp.float32)
    for i in range(RPT // 16):
        z_v[pl.ds(i * 16, 16)] = jnp.zeros((16,), jnp.float32)
    pltpu.sync_copy(dst_hbm.at[pl.ds(wid * DEG_ROWS, DEG_ROWS)], dst_v)
    pltpu.sync_copy(z_v, deg_sh.at[pl.ds(sid * RPT, RPT)])
    plsc.subcore_barrier()

    def scat(j, s):
        pltpu.async_copy(ones_v, deg_sh.at[dst_v.at[j]], sems[s], add=True)

    def dwait(s):
        pltpu.make_async_copy(ones_v, deg_sh.at[dst_v.at[0]], sems[s]).wait()

    for s in range(DSEM):
        scat(s, s)

    def body(o, carry):
        for s in range(DSEM):
            dwait(s)

            @pl.when(o < DEG_ROWS // DSEM - 1)
            def _():
                scat((o + 1) * DSEM + s, s)

        return carry

    lax.fori_loop(0, DEG_ROWS // DSEM, body, 0)
    plsc.subcore_barrier()
    pltpu.sync_copy(deg_sh.at[pl.ds(sid * RPT, RPT)],
                    degp_hbm.at[cid, pl.ds(sid * RPT, RPT)])


@functools.partial(
    pl.kernel,
    out_type=jax.ShapeDtypeStruct((NC, N_EXT, D), jnp.float32),
    mesh=_mesh(),
    scratch_types=[
        pltpu.VMEM((QROWS, CHUNK), jnp.int32),
        pltpu.VMEM((QROWS, CHUNK), jnp.int32),
        [pltpu.VMEM((CHUNK, D), jnp.float32) for _ in range(NBUF)],
        [pltpu.SemaphoreType.DMA for _ in range(NBUF)],
        [pltpu.SemaphoreType.DMA for _ in range(NBUF)],
        pltpu.VMEM_SHARED((N_EXT, D), jnp.float32),
    ],
)
def _scatter_kernel(src_hbm, dst_hbm, y_hbm, acc_hbm, src_v, dst_v, bufs, gsems,
                    ssems, acc_sh):
    cid = lax.axis_index("c")
    sid = lax.axis_index("s")
    wid = sid * NC + cid

    def gather(j, b):
        pltpu.async_copy(y_hbm.at[src_v.at[j]], bufs[b], gsems[b])

    def gwait(b):
        pltpu.make_async_copy(y_hbm.at[src_v.at[0]], bufs[b], gsems[b]).wait()

    def scatter(j, b):
        pltpu.async_copy(bufs[b], acc_sh.at[dst_v.at[j]], ssems[b], add=True)

    def swait(b):
        pltpu.make_async_copy(bufs[b], acc_sh.at[dst_v.at[0]], ssems[b]).wait()

    def ring():
        for b in range(NBUF):
            gather(b, b)

        def body(o, carry):
            for b in range(NBUF):
                j = o * NBUF + b
                gwait(b)
                scatter(j, b)
                swait(b)

                @pl.when(j + NBUF < QROWS)
                def _():
                    gather(j + NBUF, b)

            return carry

        lax.fori_loop(0, QROWS // NBUF, body, 0)

    pltpu.sync_copy(src_hbm.at[pl.ds(wid * SC_ROWS, QROWS)], src_v)
    pltpu.sync_copy(dst_hbm.at[pl.ds(wid * SC_ROWS, QROWS)], dst_v)
    pltpu.sync_copy(y_hbm.at[pl.ds(sid * IPT8, IPT8)],
                    acc_sh.at[pl.ds(sid * IPT8, IPT8)])

    @pl.when(sid == NS - 1)
    def _():
        pltpu.sync_copy(y_hbm.at[pl.ds(NS * IPT8, N - NS * IPT8)],
                        acc_sh.at[pl.ds(NS * IPT8, N - NS * IPT8)])
    plsc.subcore_barrier()
    ring()
    for seg in range(1, 4):
        pltpu.sync_copy(src_hbm.at[pl.ds(wid * SC_ROWS + seg * QROWS, QROWS)], src_v)
        pltpu.sync_copy(dst_hbm.at[pl.ds(wid * SC_ROWS + seg * QROWS, QROWS)], dst_v)
        ring()
    plsc.subcore_barrier()
    pltpu.sync_copy(acc_sh.at[pl.ds(sid * RPT, RPT)],
                    acc_hbm.at[cid, pl.ds(sid * RPT, RPT)])


RB = 1024
RBN = 1000


def _prep_body(x_ref, w_ref, d_ref, y_ref, dinv_ref):
    dinv = lax.rsqrt(d_ref[...])
    xw = jnp.dot(x_ref[...], w_ref[...], preferred_element_type=jnp.float32)
    y_ref[...] = xw * dinv
    dinv_ref[...] = dinv


def _bn_body(p0_ref, p1_ref, y_ref, dinv_ref, prm_ref, x_ref, out_ref,
             agg_vmem, st_vmem):
    ph = pl.program_id(0)
    i = pl.program_id(1)

    @pl.when(ph == 0)
    def _():
        s = p0_ref[0] + p1_ref[0] - y_ref[...]
        agg = s * dinv_ref[...] + prm_ref[0:1, :]
        agg_vmem[pl.ds(i * RBN, RBN), :] = agg
        su = jnp.sum(agg, axis=0, keepdims=True)
        sq = jnp.sum(agg * agg, axis=0, keepdims=True)
        upd = jnp.concatenate([su, sq, jnp.zeros((6, D), jnp.float32)], axis=0)

        @pl.when(i == 0)
        def _():
            st_vmem[...] = jnp.zeros((8, D), jnp.float32)

        st_vmem[...] += upd

    @pl.when(ph == 1)
    def _():
        mean = st_vmem[0:1, :] / float(N)
        ex2 = st_vmem[1:2, :] / float(N)
        var = ex2 - mean * mean
        rstd = lax.rsqrt(var + EPS)
        g = prm_ref[1:2, :]
        be = prm_ref[2:3, :]
        agg = agg_vmem[pl.ds(i * RBN, RBN), :]
        h = (agg - mean) * rstd * g + be
        out_ref[...] = jnp.maximum(h, 0.0) + x_ref[...]


def _stats_body(p0_ref, p1_ref, y_ref, dinv_ref, prm_ref, agg_ref, st_ref):
    i = pl.program_id(0)
    s = p0_ref[0] + p1_ref[0] - y_ref[...]
    b = prm_ref[0:1, :]
    agg = s * dinv_ref[...] + b
    agg_ref[...] = agg
    su = jnp.sum(agg, axis=0, keepdims=True)
    sq = jnp.sum(agg * agg, axis=0, keepdims=True)
    upd = jnp.concatenate([su, sq, jnp.zeros((6, D), jnp.float32)], axis=0)

    @pl.when(i == 0)
    def _():
        st_ref[...] = jnp.zeros((8, D), jnp.float32)

    st_ref[...] += upd


def _norm_body(agg_ref, st_ref, prm_ref, x_ref, out_ref):
    mean = st_ref[0:1, :] / float(N)
    ex2 = st_ref[1:2, :] / float(N)
    var = ex2 - mean * mean
    rstd = lax.rsqrt(var + EPS)
    g = prm_ref[1:2, :]
    be = prm_ref[2:3, :]
    h = (agg_ref[...] - mean) * rstd * g + be
    out_ref[...] = jnp.maximum(h, 0.0) + x_ref[...]


def kernel(x, edge_index, W, b, gamma, beta):
    ei = edge_index.astype(jnp.int32)
    npad = E_PAD - E
    pad_src = jnp.arange(npad, dtype=jnp.int32) % N_PAD_ROWS
    pad_dst = pad_src + N
    src_flat = jnp.concatenate([ei[0], pad_src])
    dst_flat = jnp.concatenate([ei[1], pad_dst])
    src2d = src_flat.reshape(E_PAD // CHUNK, CHUNK)
    dst2d = dst_flat.reshape(E_PAD // CHUNK, CHUNK)
    prm = jnp.zeros((8, D), jnp.float32).at[0].set(b).at[1].set(gamma).at[2].set(beta)

    degp = _deg_kernel(dst2d)
    deg = (degp[0, :N] + degp[1, :N] + 1.0).reshape(N, 1)

    y, dinv = pl.pallas_call(
        _prep_body,
        grid=(N // RBN,),
        in_specs=[
            pl.BlockSpec((RBN, D), lambda i: (i, 0)),
            pl.BlockSpec((D, D), lambda i: (0, 0)),
            pl.BlockSpec((RBN, 1), lambda i: (i, 0)),
        ],
        out_specs=[
            pl.BlockSpec((RBN, D), lambda i: (i, 0)),
            pl.BlockSpec((RBN, 1), lambda i: (i, 0)),
        ],
        out_shape=[
            jax.ShapeDtypeStruct((N, D), jnp.float32),
            jax.ShapeDtypeStruct((N, 1), jnp.float32),
        ],
    )(x, W, deg)

    acc = _scatter_kernel(src2d, dst2d, y)

    out = pl.pallas_call(
        _bn_body,
        grid=(2, N // RBN),
        in_specs=[
            pl.BlockSpec((1, RBN, D), lambda p, i: (0, i * (1 - p), 0)),
            pl.BlockSpec((1, RBN, D), lambda p, i: (1, i * (1 - p), 0)),
            pl.BlockSpec((RBN, D), lambda p, i: (i * (1 - p), 0)),
            pl.BlockSpec((RBN, 1), lambda p, i: (i * (1 - p), 0)),
            pl.BlockSpec((8, D), lambda p, i: (0, 0)),
            pl.BlockSpec((RBN, D), lambda p, i: (i * p, 0)),
        ],
        out_specs=pl.BlockSpec((RBN, D), lambda p, i: (i, 0)),
        out_shape=jax.ShapeDtypeStruct((N, D), jnp.float32),
        scratch_shapes=[
            pltpu.VMEM((N, D), jnp.float32),
            pltpu.VMEM((8, D), jnp.float32),
        ],
        compiler_params=pltpu.CompilerParams(
            dimension_semantics=("arbitrary", "arbitrary")),
    )(acc, acc, y, dinv, prm, x)

    return out
